# baseline (speedup 1.0000x reference)
import dataclasses
import functools

import jax
import jax.numpy as jnp
from jax import lax
from jax.experimental import pallas as pl
from jax.experimental.pallas import tpu as pltpu
from jax.experimental.pallas import tpu_sc as plsc

N_NODES = 10000
D_FEAT = 128
N_HIDDEN = 128
N_LABELS = 16

NC, NS = 2, 16
NW = NC * NS
WROWS = 125
WLANE = 80
EPW = WROWS * WLANE
N_PAD = 10240
SLICE = N_PAD // NS
CHUNK = 16
DST_OFF = NW * EPW

_mesh = plsc.VectorSubcoreMesh(core_axis_name="c", subcore_axis_name="s")

_sc_params = pltpu.CompilerParams()
if "needs_layout_passes" in pltpu.CompilerParams.__dataclass_fields__:
    _sc_params = dataclasses.replace(_sc_params, needs_layout_passes=False)


@functools.partial(
    pl.kernel,
    out_type=jax.ShapeDtypeStruct((NC, N_PAD), jnp.float32),
    mesh=_mesh,
    compiler_params=_sc_params,
    scratch_types=[
        pltpu.VMEM((EPW,), jnp.int32),
        pltpu.VMEM((N_PAD,), jnp.float32),
        pltpu.VMEM((NS, SLICE), jnp.float32),
        pltpu.VMEM((SLICE,), jnp.float32),
        pltpu.VMEM_SHARED((NS, N_PAD), jnp.float32),
        pltpu.SemaphoreType.DMA,
    ],
)
def _sc_degree(ei_hbm, out_hbm, idx_v, priv_v, red_v, ds_v, stage_sh, sem):
    c = lax.axis_index("c")
    s = lax.axis_index("s")
    g = c * NS + s
    base = g * EPW
    ones16 = jnp.ones((CHUNK,), jnp.float32)

    pltpu.async_copy(ei_hbm.at[pl.ds(DST_OFF + base, EPW)], idx_v, sem)

    @pl.loop(0, N_PAD // CHUNK)
    def _(i):
        priv_v[pl.ds(i * CHUNK, CHUNK)] = jnp.zeros((CHUNK,), jnp.float32)

    pltpu.make_async_copy(ei_hbm.at[pl.ds(DST_OFF + base, EPW)],
                          idx_v, sem).wait()

    @pl.loop(0, EPW // CHUNK)
    def _(i):
        plsc.addupdate_scatter(priv_v, [idx_v[pl.ds(i * CHUNK, CHUNK)]],
                               ones16)

    pltpu.sync_copy(priv_v, stage_sh.at[s])
    plsc.subcore_barrier()

    for k in range(NS):
        pltpu.async_copy(stage_sh.at[k, pl.ds(s * SLICE, SLICE)],
                         red_v.at[k], sem)
    for k in range(NS):
        pltpu.make_async_copy(stage_sh.at[k, pl.ds(s * SLICE, SLICE)],
                              red_v.at[k], sem).wait()

    @pl.loop(0, SLICE // CHUNK)
    def _(i):
        sl = pl.ds(i * CHUNK, CHUNK)
        acc = red_v[0, sl]
        for k in range(1, NS):
            acc = acc + red_v[k, sl]
        ds_v[sl] = acc

    pltpu.sync_copy(ds_v, out_hbm.at[c, pl.ds(s * SLICE, SLICE)])


def _newton_rsqrt(d):
    half = d * jnp.float32(0.5)
    yi = jnp.int32(0x5F3759DF) - lax.shift_right_logical(
        plsc.bitcast(d, jnp.int32), jnp.int32(1))
    y = plsc.bitcast(yi, jnp.float32)
    for _ in range(3):
        y = y * (jnp.float32(1.5) - half * y * y)
    return y


@functools.partial(
    pl.kernel,
    out_type=(
        jax.ShapeDtypeStruct((NC, N_PAD), jnp.float32),
        jax.ShapeDtypeStruct((N_PAD,), jnp.float32),
    ),
    mesh=_mesh,
    compiler_params=_sc_params,
    scratch_types=[
        pltpu.VMEM((SLICE,), jnp.float32),
        pltpu.VMEM((SLICE,), jnp.float32),
        pltpu.VMEM((SLICE,), jnp.float32),
        pltpu.VMEM((N_PAD,), jnp.float32),
        pltpu.VMEM((EPW,), jnp.int32),
        pltpu.VMEM((WROWS, WLANE), jnp.int32),
        pltpu.VMEM((EPW,), jnp.float32),
        pltpu.VMEM((SLICE,), jnp.float32),
        pltpu.VMEM_SHARED((N_PAD,), jnp.float32),
        pltpu.VMEM_SHARED((N_PAD,), jnp.float32),
        pltpu.SemaphoreType.DMA,
        pltpu.SemaphoreType.DMA,
    ],
)
def _sc_neighbor_sum(degp_hbm, ei_hbm, out_hbm, dinv_hbm,
                     p0_v, p1_v, ds_v, dinv_v, didx_v, sidx_v, val_v,
                     zero_v, dinv_sh, acc_sh, sem, lsem):
    c = lax.axis_index("c")
    s = lax.axis_index("s")
    g = c * NS + s
    base = g * EPW

    @pl.loop(0, WROWS)
    def _(j):
        pltpu.async_copy(ei_hbm.at[pl.ds(base + j * WLANE, WLANE)],
                         sidx_v.at[j], lsem)

    @pl.loop(0, SLICE // CHUNK)
    def _(i):
        zero_v[pl.ds(i * CHUNK, CHUNK)] = jnp.zeros((CHUNK,), jnp.float32)

    pltpu.sync_copy(zero_v, acc_sh.at[pl.ds(s * SLICE, SLICE)])
    pltpu.sync_copy(degp_hbm.at[0, pl.ds(s * SLICE, SLICE)], p0_v)
    pltpu.sync_copy(degp_hbm.at[1, pl.ds(s * SLICE, SLICE)], p1_v)
    pltpu.sync_copy(ei_hbm.at[pl.ds(DST_OFF + base, EPW)], didx_v)

    @pl.loop(0, SLICE // CHUNK)
    def _(i):
        sl = pl.ds(i * CHUNK, CHUNK)
        deg = p0_v[sl] + p1_v[sl] + jnp.float32(1.0)
        ds_v[sl] = _newton_rsqrt(deg)

    pltpu.sync_copy(ds_v, dinv_sh.at[pl.ds(s * SLICE, SLICE)])

    @pl.when(c == 0)
    def _():
        pltpu.sync_copy(ds_v, dinv_hbm.at[pl.ds(s * SLICE, SLICE)])

    plsc.subcore_barrier()
    pltpu.sync_copy(dinv_sh, dinv_v)

    @pl.loop(0, WROWS)
    def _(j):
        pltpu.make_async_copy(ei_hbm.at[pl.ds(base + j * WLANE, WLANE)],
                              sidx_v.at[j], lsem).wait()

    @pl.loop(0, WROWS)
    def _(j):
        @pl.loop(0, WLANE // CHUNK)
        def _(k):
            sl = pl.ds(j * WLANE + k * CHUNK, CHUNK)
            val_v[sl] = plsc.load_gather(dinv_v, [didx_v[sl]])

        pltpu.async_copy(val_v.at[pl.ds(j * WLANE, WLANE)],
                         acc_sh.at[sidx_v.at[j]], sem, add=True)

    @pl.loop(0, WROWS)
    def _(j):
        pltpu.make_async_copy(val_v.at[pl.ds(j * WLANE, WLANE)],
                              acc_sh.at[sidx_v.at[j]], sem).wait()

    plsc.subcore_barrier()
    pltpu.sync_copy(acc_sh.at[pl.ds(s * SLICE, SLICE)],
                    out_hbm.at[c, pl.ds(s * SLICE, SLICE)])


def _tc_head_body(dinv_ref, tp_ref, x_ref, w1_ref, b1_ref, w2_ref, b2_ref,
                  o_ref):
    dinv = dinv_ref[...]
    tp = tp_ref[...]
    cvec = dinv * (dinv + tp[0] + tp[1])
    v = lax.dot_general(
        cvec[:N_NODES].reshape(1, N_NODES), x_ref[...],
        (((1,), (0,)), ((), ())),
        precision=lax.Precision.HIGHEST,
        preferred_element_type=jnp.float32,
    )
    pooled = v @ w1_ref[...] + jnp.float32(N_NODES) * b1_ref[...].reshape(1, N_HIDDEN)
    logits = pooled @ w2_ref[...] + b2_ref[...].reshape(1, N_LABELS)
    o_ref[...] = jax.nn.sigmoid(logits)


_tc_head = pl.pallas_call(
    _tc_head_body,
    out_shape=jax.ShapeDtypeStruct((1, N_LABELS), jnp.float32),
)


def kernel(x, edge_index, W1, b1, W2, b2):
    ei = edge_index.astype(jnp.int32).reshape(-1)

    degp = _sc_degree(ei)
    tp, dinv = _sc_neighbor_sum(degp, ei)
    out = _tc_head(dinv, tp, x, W1, b1, W2, b2)
    return out.reshape(N_LABELS)


# speedup vs baseline: 1.0758x; 1.0758x over previous
import dataclasses
import functools

import jax
import jax.numpy as jnp
from jax import lax
from jax.experimental import pallas as pl
from jax.experimental.pallas import tpu as pltpu
from jax.experimental.pallas import tpu_sc as plsc

N_NODES = 10000
D_FEAT = 128
N_HIDDEN = 128
N_LABELS = 16

NC, NS = 2, 16
NW = NC * NS
WROWS = 125
WLANE = 80
EPW = WROWS * WLANE
N_PAD = 10240
SLICE = N_PAD // NS
CHUNK = 16
DST_OFF = NW * EPW

_mesh = plsc.VectorSubcoreMesh(core_axis_name="c", subcore_axis_name="s")

_sc_params = pltpu.CompilerParams()
if "needs_layout_passes" in pltpu.CompilerParams.__dataclass_fields__:
    _sc_params = dataclasses.replace(_sc_params, needs_layout_passes=False)


@functools.partial(
    pl.kernel,
    out_type=jax.ShapeDtypeStruct((NC, N_PAD), jnp.float32),
    mesh=_mesh,
    compiler_params=_sc_params,
    scratch_types=[
        pltpu.VMEM((WROWS, WLANE), jnp.int32),
        pltpu.VMEM((EPW,), jnp.float32),
        pltpu.VMEM((SLICE,), jnp.float32),
        pltpu.VMEM_SHARED((N_PAD,), jnp.float32),
        pltpu.SemaphoreType.DMA,
        pltpu.SemaphoreType.DMA,
    ],
)
def _sc_degree(ei_hbm, out_hbm, idx_v, val_v, zero_v, acc_sh, sem, lsem):
    c = lax.axis_index("c")
    s = lax.axis_index("s")
    g = c * NS + s
    base = g * EPW

    @pl.loop(0, WROWS)
    def _(j):
        pltpu.async_copy(ei_hbm.at[pl.ds(DST_OFF + base + j * WLANE, WLANE)],
                         idx_v.at[j], lsem)

    @pl.loop(0, SLICE // CHUNK)
    def _(i):
        zero_v[pl.ds(i * CHUNK, CHUNK)] = jnp.zeros((CHUNK,), jnp.float32)

    pltpu.sync_copy(zero_v, acc_sh.at[pl.ds(s * SLICE, SLICE)])

    @pl.loop(0, EPW // CHUNK)
    def _(i):
        val_v[pl.ds(i * CHUNK, CHUNK)] = jnp.ones((CHUNK,), jnp.float32)

    @pl.loop(0, WROWS)
    def _(j):
        pltpu.make_async_copy(ei_hbm.at[pl.ds(DST_OFF + base + j * WLANE, WLANE)],
                              idx_v.at[j], lsem).wait()

    plsc.subcore_barrier()

    @pl.loop(0, WROWS)
    def _(j):
        pltpu.async_copy(val_v.at[pl.ds(j * WLANE, WLANE)],
                         acc_sh.at[idx_v.at[j]], sem, add=True)

    @pl.loop(0, WROWS)
    def _(j):
        pltpu.make_async_copy(val_v.at[pl.ds(j * WLANE, WLANE)],
                              acc_sh.at[idx_v.at[j]], sem).wait()

    plsc.subcore_barrier()
    pltpu.sync_copy(acc_sh.at[pl.ds(s * SLICE, SLICE)],
                    out_hbm.at[c, pl.ds(s * SLICE, SLICE)])


def _newton_rsqrt(d):
    half = d * jnp.float32(0.5)
    yi = jnp.int32(0x5F3759DF) - lax.shift_right_logical(
        plsc.bitcast(d, jnp.int32), jnp.int32(1))
    y = plsc.bitcast(yi, jnp.float32)
    for _ in range(3):
        y = y * (jnp.float32(1.5) - half * y * y)
    return y


@functools.partial(
    pl.kernel,
    out_type=(
        jax.ShapeDtypeStruct((NC, N_PAD), jnp.float32),
        jax.ShapeDtypeStruct((N_PAD,), jnp.float32),
    ),
    mesh=_mesh,
    compiler_params=_sc_params,
    scratch_types=[
        pltpu.VMEM((SLICE,), jnp.float32),
        pltpu.VMEM((SLICE,), jnp.float32),
        pltpu.VMEM((SLICE,), jnp.float32),
        pltpu.VMEM((N_PAD,), jnp.float32),
        pltpu.VMEM((EPW,), jnp.int32),
        pltpu.VMEM((WROWS, WLANE), jnp.int32),
        pltpu.VMEM((EPW,), jnp.float32),
        pltpu.VMEM((SLICE,), jnp.float32),
        pltpu.VMEM_SHARED((N_PAD,), jnp.float32),
        pltpu.VMEM_SHARED((N_PAD,), jnp.float32),
        pltpu.SemaphoreType.DMA,
        pltpu.SemaphoreType.DMA,
    ],
)
def _sc_neighbor_sum(degp_hbm, ei_hbm, out_hbm, dinv_hbm,
                     p0_v, p1_v, ds_v, dinv_v, didx_v, sidx_v, val_v,
                     zero_v, dinv_sh, acc_sh, sem, lsem):
    c = lax.axis_index("c")
    s = lax.axis_index("s")
    g = c * NS + s
    base = g * EPW

    @pl.loop(0, WROWS)
    def _(j):
        pltpu.async_copy(ei_hbm.at[pl.ds(base + j * WLANE, WLANE)],
                         sidx_v.at[j], lsem)

    @pl.loop(0, SLICE // CHUNK)
    def _(i):
        zero_v[pl.ds(i * CHUNK, CHUNK)] = jnp.zeros((CHUNK,), jnp.float32)

    pltpu.sync_copy(zero_v, acc_sh.at[pl.ds(s * SLICE, SLICE)])
    pltpu.sync_copy(degp_hbm.at[0, pl.ds(s * SLICE, SLICE)], p0_v)
    pltpu.sync_copy(degp_hbm.at[1, pl.ds(s * SLICE, SLICE)], p1_v)
    pltpu.sync_copy(ei_hbm.at[pl.ds(DST_OFF + base, EPW)], didx_v)

    @pl.loop(0, SLICE // CHUNK)
    def _(i):
        sl = pl.ds(i * CHUNK, CHUNK)
        deg = p0_v[sl] + p1_v[sl] + jnp.float32(1.0)
        ds_v[sl] = _newton_rsqrt(deg)

    pltpu.sync_copy(ds_v, dinv_sh.at[pl.ds(s * SLICE, SLICE)])

    @pl.when(c == 0)
    def _():
        pltpu.sync_copy(ds_v, dinv_hbm.at[pl.ds(s * SLICE, SLICE)])

    plsc.subcore_barrier()
    pltpu.sync_copy(dinv_sh, dinv_v)

    @pl.loop(0, WROWS)
    def _(j):
        pltpu.make_async_copy(ei_hbm.at[pl.ds(base + j * WLANE, WLANE)],
                              sidx_v.at[j], lsem).wait()

    @pl.loop(0, WROWS)
    def _(j):
        @pl.loop(0, WLANE // CHUNK)
        def _(k):
            sl = pl.ds(j * WLANE + k * CHUNK, CHUNK)
            val_v[sl] = plsc.load_gather(dinv_v, [didx_v[sl]])

        pltpu.async_copy(val_v.at[pl.ds(j * WLANE, WLANE)],
                         acc_sh.at[sidx_v.at[j]], sem, add=True)

    @pl.loop(0, WROWS)
    def _(j):
        pltpu.make_async_copy(val_v.at[pl.ds(j * WLANE, WLANE)],
                              acc_sh.at[sidx_v.at[j]], sem).wait()

    plsc.subcore_barrier()
    pltpu.sync_copy(acc_sh.at[pl.ds(s * SLICE, SLICE)],
                    out_hbm.at[c, pl.ds(s * SLICE, SLICE)])


def _tc_head_body(dinv_ref, tp_ref, x_ref, w1_ref, b1_ref, w2_ref, b2_ref,
                  o_ref):
    dinv = dinv_ref[...]
    tp = tp_ref[...]
    cvec = dinv * (dinv + tp[0] + tp[1])
    v = lax.dot_general(
        cvec[:N_NODES].reshape(1, N_NODES), x_ref[...],
        (((1,), (0,)), ((), ())),
        preferred_element_type=jnp.float32,
    )
    pooled = v @ w1_ref[...] + jnp.float32(N_NODES) * b1_ref[...].reshape(1, N_HIDDEN)
    logits = pooled @ w2_ref[...] + b2_ref[...].reshape(1, N_LABELS)
    o_ref[...] = jax.nn.sigmoid(logits)


_tc_head = pl.pallas_call(
    _tc_head_body,
    out_shape=jax.ShapeDtypeStruct((1, N_LABELS), jnp.float32),
)


def kernel(x, edge_index, W1, b1, W2, b2):
    ei = edge_index.astype(jnp.int32).reshape(-1)

    degp = _sc_degree(ei)
    tp, dinv = _sc_neighbor_sum(degp, ei)
    out = _tc_head(dinv, tp, x, W1, b1, W2, b2)
    return out.reshape(N_LABELS)
